# Initial kernel scaffold; baseline (speedup 1.0000x reference)
import jax
import jax.numpy as jnp
from jax.experimental import pallas as pl


def kernel(x, w1, b1, w2, b2, w3, b3):
    raise NotImplementedError("write your pallas kernel here")


# speedup vs baseline: 1.0104x; 1.0104x over previous
import functools

import jax
import jax.numpy as jnp
from jax.experimental import pallas as pl
from jax.experimental.pallas import tpu as pltpu

LANE = 128
SUBLANE = 8


def _round_up(n: int, m: int) -> int:
    return ((n + m - 1) // m) * m


def _sin_mlp_kernel(x_ref, w1_ref, b1_ref, w2_ref, w3_ref, b23_ref, o_ref):
    x = x_ref[...].astype(jnp.bfloat16)
    h = jnp.dot(x, w1_ref[...], preferred_element_type=jnp.float32)
    s = jnp.sin(h + b1_ref[...]).astype(jnp.bfloat16)
    z = jnp.dot(s, w2_ref[...], preferred_element_type=jnp.float32)
    r = w3_ref.shape[0]
    y = b23_ref[...] + z[:, 0:1] * w3_ref[0:1, :]
    for j in range(1, r):
        y = y + z[:, j : j + 1] * w3_ref[j : j + 1, :]
    o_ref[...] = y


@functools.partial(jax.jit, static_argnames=("tb",))
def _forward(x, w1, b1, w2, b2, w3, b3, *, tb=512):
    B, d_in = x.shape
    H = w1.shape[1]
    r, d_out = w3.shape
    n_pad = _round_up(d_out, LANE)

    w1b = w1.astype(jnp.bfloat16)
    w2p = (
        jnp.zeros((H, LANE), jnp.bfloat16)
        .at[:, :r]
        .set(w2.astype(jnp.bfloat16))
    )
    b23 = (jnp.dot(b2, w3, preferred_element_type=jnp.float32) + b3).reshape(1, -1)
    if n_pad != d_out:
        w3 = jnp.pad(w3, ((0, 0), (0, n_pad - d_out)))
        b23 = jnp.pad(b23, ((0, 0), (0, n_pad - d_out)))
    b1r = b1.reshape(1, -1)

    TB = min(tb, _round_up(B, SUBLANE))
    B_pad = _round_up(B, TB)
    if B_pad != B:
        x = jnp.pad(x, ((0, B_pad - B), (0, 0)))

    out = pl.pallas_call(
        _sin_mlp_kernel,
        out_shape=jax.ShapeDtypeStruct((B_pad, n_pad), jnp.float32),
        grid=(B_pad // TB,),
        in_specs=[
            pl.BlockSpec((TB, d_in), lambda i: (i, 0)),
            pl.BlockSpec((d_in, H), lambda i: (0, 0)),
            pl.BlockSpec((1, H), lambda i: (0, 0)),
            pl.BlockSpec((H, LANE), lambda i: (0, 0)),
            pl.BlockSpec((r, n_pad), lambda i: (0, 0)),
            pl.BlockSpec((1, n_pad), lambda i: (0, 0)),
        ],
        out_specs=pl.BlockSpec((TB, n_pad), lambda i: (i, 0)),
        compiler_params=pltpu.CompilerParams(
            dimension_semantics=("parallel",),
        ),
    )(x, w1b, b1r, w2p, w3, b23)

    if B_pad != B or n_pad != d_out:
        out = out[:B, :d_out]
    return out


def kernel(x, w1, b1, w2, b2, w3, b3):
    return _forward(x, w1, b1, w2, b2, w3, b3)


# speedup vs baseline: 2.6457x; 2.6184x over previous
import functools

import jax
import jax.numpy as jnp
from jax.experimental import pallas as pl
from jax.experimental.pallas import tpu as pltpu

LANE = 128
SUBLANE = 8

_TWO_OVER_PI = 0.6366197723675814
_PIO2_HI = 1.57079637050628662109375
_PIO2_LO = -4.37113900018624283e-8
_S3, _S5, _S7 = -1.6666654611e-1, 8.3321608736e-3, -1.9515295891e-4
_C2, _C4, _C6, _C8 = -0.5, 4.166664568298827e-2, -1.388731625493765e-3, 2.443315711809948e-5


def _fast_sin(h):
    t = h * _TWO_OVER_PI
    kf = jax.lax.round(t, jax.lax.RoundingMethod.TO_NEAREST_EVEN)
    ki = kf.astype(jnp.int32)
    r = h - kf * _PIO2_HI
    r = r - kf * _PIO2_LO
    r2 = r * r
    ps = r + r * (r2 * (_S3 + r2 * (_S5 + r2 * _S7)))
    pc = 1.0 + r2 * (_C2 + r2 * (_C4 + r2 * (_C6 + r2 * _C8)))
    res = jnp.where(jnp.bitwise_and(ki, 1) == 1, pc, ps)
    sign = jnp.left_shift(jnp.bitwise_and(ki, 2), 30)
    bits = jax.lax.bitcast_convert_type(res, jnp.int32) ^ sign
    return jax.lax.bitcast_convert_type(bits, jnp.float32)


def _round_up(n: int, m: int) -> int:
    return ((n + m - 1) // m) * m


def _sin_mlp_kernel(x_ref, w1_ref, b1_ref, w2_ref, w3_ref, b23_ref, o_ref):
    x = x_ref[...].astype(jnp.bfloat16)
    h = jnp.dot(x, w1_ref[...], preferred_element_type=jnp.float32)
    s = _fast_sin(h + b1_ref[...]).astype(jnp.bfloat16)
    z = jnp.dot(s, w2_ref[...], preferred_element_type=jnp.float32)
    r = w3_ref.shape[0]
    y = b23_ref[...] + z[:, 0:1] * w3_ref[0:1, :]
    for j in range(1, r):
        y = y + z[:, j : j + 1] * w3_ref[j : j + 1, :]
    o_ref[...] = y


@functools.partial(jax.jit, static_argnames=("tb",))
def _forward(x, w1, b1, w2, b2, w3, b3, *, tb=512):
    B, d_in = x.shape
    H = w1.shape[1]
    r, d_out = w3.shape
    n_pad = _round_up(d_out, LANE)

    w1b = w1.astype(jnp.bfloat16)
    w2p = (
        jnp.zeros((H, LANE), jnp.bfloat16)
        .at[:, :r]
        .set(w2.astype(jnp.bfloat16))
    )
    b23 = (jnp.dot(b2, w3, preferred_element_type=jnp.float32) + b3).reshape(1, -1)
    if n_pad != d_out:
        w3 = jnp.pad(w3, ((0, 0), (0, n_pad - d_out)))
        b23 = jnp.pad(b23, ((0, 0), (0, n_pad - d_out)))
    b1r = b1.reshape(1, -1)

    TB = min(tb, _round_up(B, SUBLANE))
    B_pad = _round_up(B, TB)
    if B_pad != B:
        x = jnp.pad(x, ((0, B_pad - B), (0, 0)))

    out = pl.pallas_call(
        _sin_mlp_kernel,
        out_shape=jax.ShapeDtypeStruct((B_pad, n_pad), jnp.float32),
        grid=(B_pad // TB,),
        in_specs=[
            pl.BlockSpec((TB, d_in), lambda i: (i, 0)),
            pl.BlockSpec((d_in, H), lambda i: (0, 0)),
            pl.BlockSpec((1, H), lambda i: (0, 0)),
            pl.BlockSpec((H, LANE), lambda i: (0, 0)),
            pl.BlockSpec((r, n_pad), lambda i: (0, 0)),
            pl.BlockSpec((1, n_pad), lambda i: (0, 0)),
        ],
        out_specs=pl.BlockSpec((TB, n_pad), lambda i: (i, 0)),
        compiler_params=pltpu.CompilerParams(
            dimension_semantics=("parallel",),
        ),
    )(x, w1b, b1r, w2p, w3, b23)

    if B_pad != B or n_pad != d_out:
        out = out[:B, :d_out]
    return out


def kernel(x, w1, b1, w2, b2, w3, b3):
    return _forward(x, w1, b1, w2, b2, w3, b3)


# speedup vs baseline: 3.4835x; 1.3167x over previous
import functools

import jax
import jax.numpy as jnp
from jax.experimental import pallas as pl
from jax.experimental.pallas import tpu as pltpu

LANE = 128
SUBLANE = 8

_INV_PI = 0.3183098861837907
_PI_HI = 3.14159274101257324
_PI_LO = -8.742277657347586e-8
_S3 = -0.16666656732559204
_S5 = 0.008333017118275166
_S7 = -0.00019806627824436873
_S9 = 2.6000836896855617e-06


def _fast_sin(h):
    t = h * _INV_PI
    kf = jax.lax.round(t, jax.lax.RoundingMethod.TO_NEAREST_EVEN)
    ki = kf.astype(jnp.int32)
    r = h - kf * _PI_HI
    r = r - kf * _PI_LO
    r2 = r * r
    p = _S3 + r2 * (_S5 + r2 * (_S7 + r2 * _S9))
    s = r + r * (r2 * p)
    sign = jnp.left_shift(jnp.bitwise_and(ki, 1), 31)
    bits = jax.lax.bitcast_convert_type(s, jnp.int32) ^ sign
    return jax.lax.bitcast_convert_type(bits, jnp.float32)


def _round_up(n: int, m: int) -> int:
    return ((n + m - 1) // m) * m


def _sin_mlp_kernel(x_ref, w1_ref, b1_ref, w2_ref, w3_ref, b23_ref, o_ref):
    x = x_ref[...].astype(jnp.bfloat16)
    h = jnp.dot(x, w1_ref[...], preferred_element_type=jnp.float32)
    s = _fast_sin(h + b1_ref[...])
    z = jnp.dot(s, w2_ref[...], preferred_element_type=jnp.float32)
    r = w3_ref.shape[0]
    y = b23_ref[...] + z[:, 0:1] * w3_ref[0:1, :]
    for j in range(1, r):
        y = y + z[:, j : j + 1] * w3_ref[j : j + 1, :]
    o_ref[...] = y


@functools.partial(jax.jit, static_argnames=("tb",))
def _forward(x, w1, b1, w2, b2, w3, b3, *, tb=512):
    B, d_in = x.shape
    H = w1.shape[1]
    r, d_out = w3.shape
    n_pad = _round_up(d_out, LANE)

    w1b = w1.astype(jnp.bfloat16)
    w2p = jnp.zeros((H, LANE), jnp.float32).at[:, :r].set(w2)
    b23 = (jnp.dot(b2, w3, preferred_element_type=jnp.float32) + b3).reshape(1, -1)
    if n_pad != d_out:
        w3 = jnp.pad(w3, ((0, 0), (0, n_pad - d_out)))
        b23 = jnp.pad(b23, ((0, 0), (0, n_pad - d_out)))
    b1r = b1.reshape(1, -1)

    TB = min(tb, _round_up(B, SUBLANE))
    B_pad = _round_up(B, TB)
    if B_pad != B:
        x = jnp.pad(x, ((0, B_pad - B), (0, 0)))

    out = pl.pallas_call(
        _sin_mlp_kernel,
        out_shape=jax.ShapeDtypeStruct((B_pad, n_pad), jnp.float32),
        grid=(B_pad // TB,),
        in_specs=[
            pl.BlockSpec((TB, d_in), lambda i: (i, 0)),
            pl.BlockSpec((d_in, H), lambda i: (0, 0)),
            pl.BlockSpec((1, H), lambda i: (0, 0)),
            pl.BlockSpec((H, LANE), lambda i: (0, 0)),
            pl.BlockSpec((r, n_pad), lambda i: (0, 0)),
            pl.BlockSpec((1, n_pad), lambda i: (0, 0)),
        ],
        out_specs=pl.BlockSpec((TB, n_pad), lambda i: (i, 0)),
        compiler_params=pltpu.CompilerParams(
            dimension_semantics=("parallel",),
        ),
    )(x, w1b, b1r, w2p, w3, b23)

    if B_pad != B or n_pad != d_out:
        out = out[:B, :d_out]
    return out


def kernel(x, w1, b1, w2, b2, w3, b3):
    return _forward(x, w1, b1, w2, b2, w3, b3)


# speedup vs baseline: 4.1106x; 1.1800x over previous
import functools

import jax
import jax.numpy as jnp
from jax.experimental import pallas as pl
from jax.experimental.pallas import tpu as pltpu

LANE = 128
SUBLANE = 8

_A1 = 3.141592502593994
_A3 = -5.1677069664001465
_A5 = 2.5500314235687256
_A7 = -0.5980454683303833
_A9 = 0.07722075283527374


def _fast_sin_pi_units(t):
    kf = jax.lax.round(t, jax.lax.RoundingMethod.TO_NEAREST_EVEN)
    ki = kf.astype(jnp.int32)
    r = t - kf
    r2 = r * r
    s = r * (_A1 + r2 * (_A3 + r2 * (_A5 + r2 * (_A7 + r2 * _A9))))
    sign = jnp.left_shift(jnp.bitwise_and(ki, 1), 31)
    bits = jax.lax.bitcast_convert_type(s, jnp.int32) ^ sign
    return jax.lax.bitcast_convert_type(bits, jnp.float32)


def _round_up(n: int, m: int) -> int:
    return ((n + m - 1) // m) * m


def _sin_mlp_kernel(x_ref, w1_ref, b1_ref, w2_ref, w3_ref, b23_ref, o_ref):
    x = x_ref[...].astype(jnp.bfloat16)
    t = jnp.dot(x, w1_ref[...], preferred_element_type=jnp.float32)
    s = _fast_sin_pi_units(t + b1_ref[...])
    z = jnp.dot(s, w2_ref[...], preferred_element_type=jnp.float32)
    r = w3_ref.shape[0]
    y = b23_ref[...] + z[:, 0:1] * w3_ref[0:1, :]
    for j in range(1, r):
        y = y + z[:, j : j + 1] * w3_ref[j : j + 1, :]
    o_ref[...] = y


@functools.partial(jax.jit, static_argnames=("tb",))
def _forward(x, w1, b1, w2, b2, w3, b3, *, tb=512):
    B, d_in = x.shape
    H = w1.shape[1]
    r, d_out = w3.shape
    n_pad = _round_up(d_out, LANE)

    inv_pi = 1.0 / jnp.pi
    w1b = (w1 * inv_pi).astype(jnp.bfloat16)
    w2p = jnp.zeros((H, LANE), jnp.float32).at[:, :r].set(w2)
    b23 = (jnp.dot(b2, w3, preferred_element_type=jnp.float32) + b3).reshape(1, -1)
    if n_pad != d_out:
        w3 = jnp.pad(w3, ((0, 0), (0, n_pad - d_out)))
        b23 = jnp.pad(b23, ((0, 0), (0, n_pad - d_out)))
    b1r = (b1 * inv_pi).reshape(1, -1)

    TB = min(tb, _round_up(B, SUBLANE))
    B_pad = _round_up(B, TB)
    if B_pad != B:
        x = jnp.pad(x, ((0, B_pad - B), (0, 0)))

    out = pl.pallas_call(
        _sin_mlp_kernel,
        out_shape=jax.ShapeDtypeStruct((B_pad, n_pad), jnp.float32),
        grid=(B_pad // TB,),
        in_specs=[
            pl.BlockSpec((TB, d_in), lambda i: (i, 0)),
            pl.BlockSpec((d_in, H), lambda i: (0, 0)),
            pl.BlockSpec((1, H), lambda i: (0, 0)),
            pl.BlockSpec((H, LANE), lambda i: (0, 0)),
            pl.BlockSpec((r, n_pad), lambda i: (0, 0)),
            pl.BlockSpec((1, n_pad), lambda i: (0, 0)),
        ],
        out_specs=pl.BlockSpec((TB, n_pad), lambda i: (i, 0)),
        compiler_params=pltpu.CompilerParams(
            dimension_semantics=("parallel",),
        ),
    )(x, w1b, b1r, w2p, w3, b23)

    if B_pad != B or n_pad != d_out:
        out = out[:B, :d_out]
    return out


def kernel(x, w1, b1, w2, b2, w3, b3):
    return _forward(x, w1, b1, w2, b2, w3, b3)


# speedup vs baseline: 4.4761x; 1.0889x over previous
import functools

import jax
import jax.numpy as jnp
from jax.experimental import pallas as pl
from jax.experimental.pallas import tpu as pltpu

LANE = 128
SUBLANE = 8

_A1 = 3.141592502593994
_A3 = -5.1677069664001465
_A5 = 2.5500314235687256
_A7 = -0.5980454683303833
_A9 = 0.07722075283527374


def _fast_sin_pi_units(t):
    kf = jax.lax.round(t, jax.lax.RoundingMethod.TO_NEAREST_EVEN)
    ki = kf.astype(jnp.int32)
    r = t - kf
    r2 = r * r
    s = r * (_A1 + r2 * (_A3 + r2 * (_A5 + r2 * (_A7 + r2 * _A9))))
    sign = jnp.left_shift(jnp.bitwise_and(ki, 1), 31)
    bits = jax.lax.bitcast_convert_type(s, jnp.int32) ^ sign
    return jax.lax.bitcast_convert_type(bits, jnp.float32)


def _round_up(n: int, m: int) -> int:
    return ((n + m - 1) // m) * m


def _sin_mlp_kernel(x_ref, w1_ref, b1_ref, w2_ref, w3_ref, b23_ref, o_ref):
    x = x_ref[...].astype(jnp.bfloat16)
    t = jnp.dot(x, w1_ref[...], preferred_element_type=jnp.float32)
    s = _fast_sin_pi_units(t + b1_ref[...])
    z = jnp.dot(s, w2_ref[...], preferred_element_type=jnp.float32)
    r = w3_ref.shape[0]
    y = b23_ref[...] + z[:, 0:1] * w3_ref[0:1, :]
    for j in range(1, r):
        y = y + z[:, j : j + 1] * w3_ref[j : j + 1, :]
    o_ref[...] = y


@functools.partial(jax.jit, static_argnames=("tb",))
def _forward(x, w1, b1, w2, b2, w3, b3, *, tb=1024):
    B, d_in = x.shape
    H = w1.shape[1]
    r, d_out = w3.shape
    n_pad = _round_up(d_out, LANE)

    inv_pi = 1.0 / jnp.pi
    w1b = (w1 * inv_pi).astype(jnp.bfloat16)
    w2p = jnp.zeros((H, LANE), jnp.float32).at[:, :r].set(w2)
    b23 = (jnp.dot(b2, w3, preferred_element_type=jnp.float32) + b3).reshape(1, -1)
    if n_pad != d_out:
        w3 = jnp.pad(w3, ((0, 0), (0, n_pad - d_out)))
        b23 = jnp.pad(b23, ((0, 0), (0, n_pad - d_out)))
    b1r = (b1 * inv_pi).reshape(1, -1)

    TB = min(tb, _round_up(B, SUBLANE))
    B_pad = _round_up(B, TB)
    if B_pad != B:
        x = jnp.pad(x, ((0, B_pad - B), (0, 0)))

    out = pl.pallas_call(
        _sin_mlp_kernel,
        out_shape=jax.ShapeDtypeStruct((B_pad, n_pad), jnp.float32),
        grid=(B_pad // TB,),
        in_specs=[
            pl.BlockSpec((TB, d_in), lambda i: (i, 0)),
            pl.BlockSpec((d_in, H), lambda i: (0, 0)),
            pl.BlockSpec((1, H), lambda i: (0, 0)),
            pl.BlockSpec((H, LANE), lambda i: (0, 0)),
            pl.BlockSpec((r, n_pad), lambda i: (0, 0)),
            pl.BlockSpec((1, n_pad), lambda i: (0, 0)),
        ],
        out_specs=pl.BlockSpec((TB, n_pad), lambda i: (i, 0)),
        compiler_params=pltpu.CompilerParams(
            dimension_semantics=("parallel",),
        ),
    )(x, w1b, b1r, w2p, w3, b23)

    if B_pad != B or n_pad != d_out:
        out = out[:B, :d_out]
    return out


def kernel(x, w1, b1, w2, b2, w3, b3):
    return _forward(x, w1, b1, w2, b2, w3, b3)
